# baseline (speedup 1.0000x reference)
import functools

import jax
import jax.numpy as jnp
from jax import lax
from jax.experimental import pallas as pl
from jax.experimental.pallas import tpu as pltpu
from jax.experimental.pallas import tpu_sc as plsc

N = 32768
D = 256
D_HID = 512
D_OUT = 128
NUM_MOTIFS = 64

NC = 2
NS = 16
NW = NC * NS
L = 16
NJ = D // L

N_TC = 28672
N_SC = N - N_TC
ROWS_PER_W = N_SC // NW
CHUNK = 128
NCHUNK = ROWS_PER_W // CHUNK
GPC = CHUNK // L


def _sc_segment_sum(feats_hbm, smask_hbm, motif_hbm, out_hbm,
                    buf, smask_v, motif_v, acc, run, cur_m_ref,
                    sem0, sem1):
    cid = lax.axis_index("c")
    sid = lax.axis_index("s")
    wid = sid * NC + cid
    base = N_TC + wid * ROWS_PER_W

    pltpu.async_copy(feats_hbm.at[pl.ds(base, CHUNK), :],
                     buf.at[pl.ds(0, CHUNK), :], sem0)
    pltpu.sync_copy(smask_hbm.at[pl.ds(base, ROWS_PER_W)],
                    smask_v.at[pl.ds(0, ROWS_PER_W)])
    pltpu.sync_copy(motif_hbm.at[pl.ds(base, ROWS_PER_W)],
                    motif_v.at[pl.ds(0, ROWS_PER_W)])

    zeros = jnp.zeros((L,), jnp.float32)
    for j in range(NJ):
        run[pl.ds(j * L, L)] = zeros

    def zero_row(i, carry):
        for j in range(NJ):
            acc[i, pl.ds(j * L, L)] = zeros
        return carry
    lax.fori_loop(0, NUM_MOTIFS, zero_row, 0)

    cur_m_ref[0] = motif_v[pl.ds(0, L)][0]

    def accumulate_rows(roff, goff, nrows):
        init = tuple(zeros for _ in range(NJ))

        def row_body(r, regs):
            sval = smask_v[pl.ds(goff + r, L)]
            s = sval[0]
            return tuple(
                regs[j] + buf[roff + r, pl.ds(j * L, L)] * s
                for j in range(NJ))

        regs = plsc.parallel_loop(0, nrows, carry=init, unroll=2)(row_body)
        for j in range(NJ):
            plsc.addupdate(run.at[pl.ds(j * L, L)], regs[j])

    def chunk_body(c, carry):
        par = lax.rem(c, 2)
        roff = par * CHUNK
        coff = c * CHUNK
        dst = buf.at[pl.ds(roff, CHUNK), :]
        src = feats_hbm.at[pl.ds(base + coff, CHUNK), :]

        @pl.when(par == 0)
        def _():
            pltpu.make_async_copy(src, dst, sem0).wait()

        @pl.when(par == 1)
        def _():
            pltpu.make_async_copy(src, dst, sem1).wait()

        @pl.when(c + 1 < NCHUNK)
        def _():
            nroff = (CHUNK - roff)
            nsrc = feats_hbm.at[pl.ds(base + coff + CHUNK, CHUNK), :]
            ndst = buf.at[pl.ds(nroff, CHUNK), :]

            @pl.when(par == 1)
            def _():
                pltpu.async_copy(nsrc, ndst, sem0)

            @pl.when(par == 0)
            def _():
                pltpu.async_copy(nsrc, ndst, sem1)

        chunk_last = motif_v[pl.ds(coff + CHUNK - L, L)][L - 1]
        chunk_fast = chunk_last == cur_m_ref[0]

        @pl.when(chunk_fast)
        def _chunk_fast():
            accumulate_rows(roff, coff, CHUNK)

        @pl.when(jnp.logical_not(chunk_fast))
        def _chunk_slow():
            def group_body(g, gc):
                goff = coff + g * L
                mv = motif_v[pl.ds(goff, L)]
                group_fast = mv[L - 1] == cur_m_ref[0]

                @pl.when(group_fast)
                def _group_fast():
                    accumulate_rows(roff + g * L, goff, L)

                @pl.when(jnp.logical_not(group_fast))
                def _group_slow():
                    sv = smask_v[pl.ds(goff, L)]
                    for r in range(L):
                        m = mv[r]
                        s = sv[r]
                        row = roff + g * L + r

                        @pl.when(m != cur_m_ref[0])
                        def _flush():
                            cm = cur_m_ref[0]
                            for j in range(NJ):
                                acc[cm, pl.ds(j * L, L)] = (
                                    run[pl.ds(j * L, L)])
                                run[pl.ds(j * L, L)] = zeros
                            cur_m_ref[0] = m

                        for j in range(NJ):
                            plsc.addupdate(
                                run.at[pl.ds(j * L, L)],
                                buf[row, pl.ds(j * L, L)] * s)
                return gc
            lax.fori_loop(0, GPC, group_body, 0)
        return carry

    lax.fori_loop(0, NCHUNK, chunk_body, 0)

    cm = cur_m_ref[0]
    for j in range(NJ):
        acc[cm, pl.ds(j * L, L)] = run[pl.ds(j * L, L)]

    pltpu.sync_copy(acc, out_hbm.at[wid])


_sc_call = functools.partial(
    pl.kernel,
    mesh=plsc.VectorSubcoreMesh(core_axis_name="c", subcore_axis_name="s"),
    out_type=jax.ShapeDtypeStruct((NW, NUM_MOTIFS, D), jnp.float32),
    scratch_types=[
        pltpu.VMEM((2 * CHUNK, D), jnp.float32),
        pltpu.VMEM((ROWS_PER_W + L,), jnp.float32),
        pltpu.VMEM((ROWS_PER_W + L,), jnp.int32),
        pltpu.VMEM((NUM_MOTIFS, D), jnp.float32),
        pltpu.VMEM((D,), jnp.float32),
        pltpu.SMEM((1,), jnp.int32),
        pltpu.SemaphoreType.DMA,
        pltpu.SemaphoreType.DMA,
    ],
)(_sc_segment_sum)


TC_BLK = 4096


def _tc_onehot_partial(feats_ref, smask_ref, motif_ref, out_ref):
    i = pl.program_id(0)
    ids = lax.broadcasted_iota(jnp.int32, (NUM_MOTIFS, TC_BLK), 0)
    oh = jnp.where(ids == motif_ref[...][None, :],
                   smask_ref[...][None, :], 0.0)
    part = jnp.dot(oh, feats_ref[...], preferred_element_type=jnp.float32)

    @pl.when(i == 0)
    def _():
        out_ref[...] = part

    @pl.when(i > 0)
    def _():
        out_ref[...] += part


def _tc_readout(partials_ref, tc_part_ref, wf_ref, bf_ref, wo_ref, bo_ref,
                out_ref):
    seg = (jnp.sum(partials_ref[...], axis=0) + tc_part_ref[...])[1:]
    h = jnp.dot(seg, wf_ref[...], preferred_element_type=jnp.float32)
    h = h + bf_ref[...]
    o = jnp.dot(h, wo_ref[...], preferred_element_type=jnp.float32)
    out_ref[...] = o + bo_ref[...]


def kernel(node_feats, smask_full, motif_batch, W_feat, b_feat, W_out, b_out):
    partials = _sc_call(node_feats, smask_full, motif_batch)
    tc_part = pl.pallas_call(
        _tc_onehot_partial,
        grid=(N_TC // TC_BLK,),
        in_specs=[
            pl.BlockSpec((TC_BLK, D), lambda i: (i, 0)),
            pl.BlockSpec((TC_BLK,), lambda i: (i,)),
            pl.BlockSpec((TC_BLK,), lambda i: (i,)),
        ],
        out_specs=pl.BlockSpec((NUM_MOTIFS, D), lambda i: (0, 0)),
        out_shape=jax.ShapeDtypeStruct((NUM_MOTIFS, D), jnp.float32),
    )(node_feats, smask_full, motif_batch)
    return pl.pallas_call(
        _tc_readout,
        out_shape=jax.ShapeDtypeStruct((NUM_MOTIFS - 1, D_OUT), jnp.float32),
    )(partials, tc_part, W_feat, b_feat.reshape(1, D_HID), W_out,
      b_out.reshape(1, D_OUT))


# speedup vs baseline: 1.0165x; 1.0165x over previous
import functools

import jax
import jax.numpy as jnp
from jax import lax
from jax.experimental import pallas as pl
from jax.experimental.pallas import tpu as pltpu
from jax.experimental.pallas import tpu_sc as plsc

N = 32768
D = 256
D_HID = 512
D_OUT = 128
NUM_MOTIFS = 64

NC = 2
NS = 16
NW = NC * NS
L = 16
NJ = D // L

N_TC = 30720
N_SC = N - N_TC
ROWS_PER_W = N_SC // NW
CHUNK = 64
NCHUNK = ROWS_PER_W // CHUNK
GPC = CHUNK // L


def _sc_segment_sum(feats_hbm, smask_hbm, motif_hbm, out_hbm,
                    buf, smask_v, motif_v, acc, run, cur_m_ref, sem0):
    cid = lax.axis_index("c")
    sid = lax.axis_index("s")
    wid = sid * NC + cid
    base = N_TC + wid * ROWS_PER_W

    pltpu.async_copy(feats_hbm.at[pl.ds(base, CHUNK), :],
                     buf.at[pl.ds(0, CHUNK), :], sem0)
    pltpu.sync_copy(smask_hbm.at[pl.ds(base, ROWS_PER_W)],
                    smask_v.at[pl.ds(0, ROWS_PER_W)])
    pltpu.sync_copy(motif_hbm.at[pl.ds(base, ROWS_PER_W)],
                    motif_v.at[pl.ds(0, ROWS_PER_W)])

    zeros = jnp.zeros((L,), jnp.float32)
    for j in range(NJ):
        run[pl.ds(j * L, L)] = zeros

    def zero_row(i, carry):
        for j in range(NJ):
            acc[i, pl.ds(j * L, L)] = zeros
        return carry
    lax.fori_loop(0, NUM_MOTIFS, zero_row, 0)

    cur_m_ref[0] = motif_v[pl.ds(0, L)][0]

    def accumulate_rows(roff, goff, nrows):
        init = tuple(zeros for _ in range(NJ))

        def row_body(r, regs):
            sval = smask_v[pl.ds(goff + r, L)]
            s = sval[0]
            return tuple(
                regs[j] + buf[roff + r, pl.ds(j * L, L)] * s
                for j in range(NJ))

        regs = plsc.parallel_loop(0, nrows, carry=init)(row_body)
        for j in range(NJ):
            plsc.addupdate(run.at[pl.ds(j * L, L)], regs[j])

    def process_chunk(roff, coff):
        chunk_last = motif_v[pl.ds(coff + CHUNK - L, L)][L - 1]
        chunk_fast = chunk_last == cur_m_ref[0]

        @pl.when(chunk_fast)
        def _chunk_fast():
            accumulate_rows(roff, coff, CHUNK)

        @pl.when(jnp.logical_not(chunk_fast))
        def _chunk_slow():
            def group_body(g, gc):
                goff = coff + g * L
                mv = motif_v[pl.ds(goff, L)]
                group_fast = mv[L - 1] == cur_m_ref[0]

                @pl.when(group_fast)
                def _group_fast():
                    accumulate_rows(roff + g * L, goff, L)

                @pl.when(jnp.logical_not(group_fast))
                def _group_slow():
                    sv = smask_v[pl.ds(goff, L)]
                    for r in range(L):
                        m = mv[r]
                        s = sv[r]
                        row = roff + g * L + r

                        @pl.when(m != cur_m_ref[0])
                        def _flush():
                            cm = cur_m_ref[0]
                            for j in range(NJ):
                                acc[cm, pl.ds(j * L, L)] = (
                                    run[pl.ds(j * L, L)])
                                run[pl.ds(j * L, L)] = zeros
                            cur_m_ref[0] = m

                        for j in range(NJ):
                            plsc.addupdate(
                                run.at[pl.ds(j * L, L)],
                                buf[row, pl.ds(j * L, L)] * s)
                return gc
            lax.fori_loop(0, GPC, group_body, 0)

    pltpu.make_async_copy(feats_hbm.at[pl.ds(base, CHUNK), :],
                          buf.at[pl.ds(0, CHUNK), :], sem0).wait()
    process_chunk(0, 0)

    cm = cur_m_ref[0]
    for j in range(NJ):
        acc[cm, pl.ds(j * L, L)] = run[pl.ds(j * L, L)]

    pltpu.sync_copy(acc, out_hbm.at[wid])


_sc_call = functools.partial(
    pl.kernel,
    mesh=plsc.VectorSubcoreMesh(core_axis_name="c", subcore_axis_name="s"),
    out_type=jax.ShapeDtypeStruct((NW, NUM_MOTIFS, D), jnp.float32),
    scratch_types=[
        pltpu.VMEM((CHUNK, D), jnp.float32),
        pltpu.VMEM((ROWS_PER_W + L,), jnp.float32),
        pltpu.VMEM((ROWS_PER_W + L,), jnp.int32),
        pltpu.VMEM((NUM_MOTIFS, D), jnp.float32),
        pltpu.VMEM((D,), jnp.float32),
        pltpu.SMEM((1,), jnp.int32),
        pltpu.SemaphoreType.DMA,
    ],
)(_sc_segment_sum)


TC_BLK = 3072


def _tc_onehot_partial(feats_ref, smask_ref, motif_ref, out_ref):
    i = pl.program_id(0)
    ids = lax.broadcasted_iota(jnp.int32, (NUM_MOTIFS, TC_BLK), 0)
    oh = jnp.where(ids == motif_ref[...][None, :],
                   smask_ref[...][None, :], 0.0)
    part = jnp.dot(oh, feats_ref[...], preferred_element_type=jnp.float32)

    @pl.when(i == 0)
    def _():
        out_ref[...] = part

    @pl.when(i > 0)
    def _():
        out_ref[...] += part


def _tc_readout(partials_ref, tc_part_ref, wf_ref, bf_ref, wo_ref, bo_ref,
                out_ref):
    seg = (jnp.sum(partials_ref[...], axis=0) + tc_part_ref[...])[1:]
    h = jnp.dot(seg, wf_ref[...], preferred_element_type=jnp.float32)
    h = h + bf_ref[...]
    o = jnp.dot(h, wo_ref[...], preferred_element_type=jnp.float32)
    out_ref[...] = o + bo_ref[...]


def kernel(node_feats, smask_full, motif_batch, W_feat, b_feat, W_out, b_out):
    partials = _sc_call(node_feats, smask_full, motif_batch)
    tc_part = pl.pallas_call(
        _tc_onehot_partial,
        grid=(N_TC // TC_BLK,),
        in_specs=[
            pl.BlockSpec((TC_BLK, D), lambda i: (i, 0)),
            pl.BlockSpec((TC_BLK,), lambda i: (i,)),
            pl.BlockSpec((TC_BLK,), lambda i: (i,)),
        ],
        out_specs=pl.BlockSpec((NUM_MOTIFS, D), lambda i: (0, 0)),
        out_shape=jax.ShapeDtypeStruct((NUM_MOTIFS, D), jnp.float32),
    )(node_feats, smask_full, motif_batch)
    return pl.pallas_call(
        _tc_readout,
        out_shape=jax.ShapeDtypeStruct((NUM_MOTIFS - 1, D_OUT), jnp.float32),
    )(partials, tc_part, W_feat, b_feat.reshape(1, D_HID), W_out,
      b_out.reshape(1, D_OUT))


# speedup vs baseline: 1.0234x; 1.0068x over previous
import functools

import jax
import jax.numpy as jnp
from jax import lax
from jax.experimental import pallas as pl
from jax.experimental.pallas import tpu as pltpu
from jax.experimental.pallas import tpu_sc as plsc

N = 32768
D = 256
D_HID = 512
D_OUT = 128
NUM_MOTIFS = 64

NC = 2
NS = 16
NW = NC * NS
L = 16
NJ = D // L

N_TC = 30720
N_SC = N - N_TC
ROWS_PER_W = N_SC // NW
CHUNK = 64
NCHUNK = ROWS_PER_W // CHUNK
GPC = CHUNK // L


def _sc_segment_sum(feats_hbm, smask_hbm, motif_hbm, out_hbm,
                    buf, smask_v, motif_v, acc, run, cur_m_ref, sem0):
    cid = lax.axis_index("c")
    sid = lax.axis_index("s")
    wid = sid * NC + cid
    base = N_TC + wid * ROWS_PER_W

    pltpu.async_copy(feats_hbm.at[pl.ds(base, CHUNK), :],
                     buf.at[pl.ds(0, CHUNK), :], sem0)
    pltpu.sync_copy(smask_hbm.at[pl.ds(base, ROWS_PER_W)],
                    smask_v.at[pl.ds(0, ROWS_PER_W)])
    pltpu.sync_copy(motif_hbm.at[pl.ds(base, ROWS_PER_W)],
                    motif_v.at[pl.ds(0, ROWS_PER_W)])

    zeros = jnp.zeros((L,), jnp.float32)
    for j in range(NJ):
        run[pl.ds(j * L, L)] = zeros

    def zero_row(i):
        for j in range(NJ):
            acc[i, pl.ds(j * L, L)] = zeros
    plsc.parallel_loop(0, NUM_MOTIFS)(zero_row)

    cur_m_ref[0] = motif_v[pl.ds(0, L)][0]

    def accumulate_rows(roff, goff, nrows):
        init = tuple(zeros for _ in range(NJ))

        def row_body(r, regs):
            sval = smask_v[pl.ds(goff + r, L)]
            s = sval[0]
            return tuple(
                regs[j] + buf[roff + r, pl.ds(j * L, L)] * s
                for j in range(NJ))

        regs = plsc.parallel_loop(0, nrows, carry=init)(row_body)
        for j in range(NJ):
            plsc.addupdate(run.at[pl.ds(j * L, L)], regs[j])

    def process_chunk(roff, coff):
        chunk_last = motif_v[pl.ds(coff + CHUNK - L, L)][L - 1]
        chunk_fast = chunk_last == cur_m_ref[0]

        @pl.when(chunk_fast)
        def _chunk_fast():
            accumulate_rows(roff, coff, CHUNK)

        @pl.when(jnp.logical_not(chunk_fast))
        def _chunk_slow():
            def group_body(g, gc):
                goff = coff + g * L
                mv = motif_v[pl.ds(goff, L)]
                group_fast = mv[L - 1] == cur_m_ref[0]

                @pl.when(group_fast)
                def _group_fast():
                    accumulate_rows(roff + g * L, goff, L)

                @pl.when(jnp.logical_not(group_fast))
                def _group_slow():
                    def row_slow(r, rc):
                        m = motif_v[pl.ds(goff + r, L)][0]
                        s = smask_v[pl.ds(goff + r, L)][0]
                        row = roff + g * L + r

                        @pl.when(m != cur_m_ref[0])
                        def _flush():
                            cm = cur_m_ref[0]
                            for j in range(NJ):
                                acc[cm, pl.ds(j * L, L)] = (
                                    run[pl.ds(j * L, L)])
                                run[pl.ds(j * L, L)] = zeros
                            cur_m_ref[0] = m

                        for j in range(NJ):
                            plsc.addupdate(
                                run.at[pl.ds(j * L, L)],
                                buf[row, pl.ds(j * L, L)] * s)
                        return rc
                    lax.fori_loop(0, L, row_slow, 0)
                return gc
            lax.fori_loop(0, GPC, group_body, 0)

    pltpu.make_async_copy(feats_hbm.at[pl.ds(base, CHUNK), :],
                          buf.at[pl.ds(0, CHUNK), :], sem0).wait()
    process_chunk(0, 0)

    cm = cur_m_ref[0]
    for j in range(NJ):
        acc[cm, pl.ds(j * L, L)] = run[pl.ds(j * L, L)]

    pltpu.sync_copy(acc, out_hbm.at[wid])


_sc_call = functools.partial(
    pl.kernel,
    mesh=plsc.VectorSubcoreMesh(core_axis_name="c", subcore_axis_name="s"),
    out_type=jax.ShapeDtypeStruct((NW, NUM_MOTIFS, D), jnp.float32),
    scratch_types=[
        pltpu.VMEM((CHUNK, D), jnp.float32),
        pltpu.VMEM((ROWS_PER_W + L,), jnp.float32),
        pltpu.VMEM((ROWS_PER_W + L,), jnp.int32),
        pltpu.VMEM((NUM_MOTIFS, D), jnp.float32),
        pltpu.VMEM((D,), jnp.float32),
        pltpu.SMEM((1,), jnp.int32),
        pltpu.SemaphoreType.DMA,
    ],
)(_sc_segment_sum)


TC_BLK = 3072


def _tc_onehot_partial(feats_ref, smask_ref, motif_ref, out_ref):
    i = pl.program_id(0)
    ids = lax.broadcasted_iota(jnp.int32, (NUM_MOTIFS, TC_BLK), 0)
    oh = jnp.where(ids == motif_ref[...][None, :],
                   smask_ref[...][None, :], 0.0)
    part = jnp.dot(oh, feats_ref[...], preferred_element_type=jnp.float32)

    @pl.when(i == 0)
    def _():
        out_ref[...] = part

    @pl.when(i > 0)
    def _():
        out_ref[...] += part


def _tc_readout(partials_ref, tc_part_ref, wf_ref, bf_ref, wo_ref, bo_ref,
                out_ref):
    seg = (jnp.sum(partials_ref[...], axis=0) + tc_part_ref[...])[1:]
    h = jnp.dot(seg, wf_ref[...], preferred_element_type=jnp.float32)
    h = h + bf_ref[...]
    o = jnp.dot(h, wo_ref[...], preferred_element_type=jnp.float32)
    out_ref[...] = o + bo_ref[...]


def kernel(node_feats, smask_full, motif_batch, W_feat, b_feat, W_out, b_out):
    partials = _sc_call(node_feats, smask_full, motif_batch)
    tc_part = pl.pallas_call(
        _tc_onehot_partial,
        grid=(N_TC // TC_BLK,),
        in_specs=[
            pl.BlockSpec((TC_BLK, D), lambda i: (i, 0)),
            pl.BlockSpec((TC_BLK,), lambda i: (i,)),
            pl.BlockSpec((TC_BLK,), lambda i: (i,)),
        ],
        out_specs=pl.BlockSpec((NUM_MOTIFS, D), lambda i: (0, 0)),
        out_shape=jax.ShapeDtypeStruct((NUM_MOTIFS, D), jnp.float32),
    )(node_feats, smask_full, motif_batch)
    return pl.pallas_call(
        _tc_readout,
        out_shape=jax.ShapeDtypeStruct((NUM_MOTIFS - 1, D_OUT), jnp.float32),
    )(partials, tc_part, W_feat, b_feat.reshape(1, D_HID), W_out,
      b_out.reshape(1, D_OUT))


# speedup vs baseline: 1.0605x; 1.0363x over previous
import functools

import jax
import jax.numpy as jnp
from jax import lax
from jax.experimental import pallas as pl
from jax.experimental.pallas import tpu as pltpu
from jax.experimental.pallas import tpu_sc as plsc

N = 32768
D = 256
D_HID = 512
D_OUT = 128
NUM_MOTIFS = 64

NC = 2
NS = 16
NW = NC * NS
L = 16
NJ = D // L

N_TC = 28672
N_SC = N - N_TC
ROWS_PER_W = N_SC // NW
CHUNK = 128
NCHUNK = ROWS_PER_W // CHUNK
GPC = CHUNK // L


def _sc_segment_sum(feats_hbm, smask_hbm, motif_hbm, out_hbm,
                    buf, smask_v, motif_v, acc, run, cur_m_ref, sem0):
    cid = lax.axis_index("c")
    sid = lax.axis_index("s")
    wid = sid * NC + cid
    base = N_TC + wid * ROWS_PER_W

    pltpu.async_copy(feats_hbm.at[pl.ds(base, CHUNK), :],
                     buf.at[pl.ds(0, CHUNK), :], sem0)
    pltpu.sync_copy(smask_hbm.at[pl.ds(base, ROWS_PER_W)],
                    smask_v.at[pl.ds(0, ROWS_PER_W)])
    pltpu.sync_copy(motif_hbm.at[pl.ds(base, ROWS_PER_W)],
                    motif_v.at[pl.ds(0, ROWS_PER_W)])

    zeros = jnp.zeros((L,), jnp.float32)
    for j in range(NJ):
        run[pl.ds(j * L, L)] = zeros

    def zero_row(i):
        for j in range(NJ):
            acc[i, pl.ds(j * L, L)] = zeros
    plsc.parallel_loop(0, NUM_MOTIFS)(zero_row)

    cur_m_ref[0] = motif_v[pl.ds(0, L)][0]

    def accumulate_rows(roff, goff, nrows):
        init = tuple(zeros for _ in range(NJ))

        def row_body(r, regs):
            sval = smask_v[pl.ds(goff + r, L)]
            s = sval[0]
            return tuple(
                regs[j] + buf[roff + r, pl.ds(j * L, L)] * s
                for j in range(NJ))

        regs = plsc.parallel_loop(0, nrows, carry=init)(row_body)
        for j in range(NJ):
            plsc.addupdate(run.at[pl.ds(j * L, L)], regs[j])

    def process_chunk(roff, coff):
        chunk_last = motif_v[pl.ds(coff + CHUNK - L, L)][L - 1]
        chunk_fast = chunk_last == cur_m_ref[0]

        @pl.when(chunk_fast)
        def _chunk_fast():
            accumulate_rows(roff, coff, CHUNK)

        @pl.when(jnp.logical_not(chunk_fast))
        def _chunk_slow():
            def group_body(g, gc):
                goff = coff + g * L
                mv = motif_v[pl.ds(goff, L)]
                group_fast = mv[L - 1] == cur_m_ref[0]

                @pl.when(group_fast)
                def _group_fast():
                    accumulate_rows(roff + g * L, goff, L)

                @pl.when(jnp.logical_not(group_fast))
                def _group_slow():
                    def row_slow(r, rc):
                        m = motif_v[pl.ds(goff + r, L)][0]
                        s = smask_v[pl.ds(goff + r, L)][0]
                        row = roff + g * L + r

                        @pl.when(m != cur_m_ref[0])
                        def _flush():
                            cm = cur_m_ref[0]
                            for j in range(NJ):
                                acc[cm, pl.ds(j * L, L)] = (
                                    run[pl.ds(j * L, L)])
                                run[pl.ds(j * L, L)] = zeros
                            cur_m_ref[0] = m

                        for j in range(NJ):
                            plsc.addupdate(
                                run.at[pl.ds(j * L, L)],
                                buf[row, pl.ds(j * L, L)] * s)
                        return rc
                    lax.fori_loop(0, L, row_slow, 0)
                return gc
            lax.fori_loop(0, GPC, group_body, 0)

    pltpu.make_async_copy(feats_hbm.at[pl.ds(base, CHUNK), :],
                          buf.at[pl.ds(0, CHUNK), :], sem0).wait()
    process_chunk(0, 0)

    cm = cur_m_ref[0]
    for j in range(NJ):
        acc[cm, pl.ds(j * L, L)] = run[pl.ds(j * L, L)]

    pltpu.sync_copy(acc, out_hbm.at[wid])


_sc_call = functools.partial(
    pl.kernel,
    mesh=plsc.VectorSubcoreMesh(core_axis_name="c", subcore_axis_name="s"),
    out_type=jax.ShapeDtypeStruct((NW, NUM_MOTIFS, D), jnp.float32),
    scratch_types=[
        pltpu.VMEM((CHUNK, D), jnp.float32),
        pltpu.VMEM((ROWS_PER_W + L,), jnp.float32),
        pltpu.VMEM((ROWS_PER_W + L,), jnp.int32),
        pltpu.VMEM((NUM_MOTIFS, D), jnp.float32),
        pltpu.VMEM((D,), jnp.float32),
        pltpu.SMEM((1,), jnp.int32),
        pltpu.SemaphoreType.DMA,
    ],
)(_sc_segment_sum)


TC_BLK = 4096


def _tc_onehot_partial(feats_ref, smask_ref, motif_ref, out_ref):
    i = pl.program_id(0)
    ids = lax.broadcasted_iota(jnp.int32, (NUM_MOTIFS, TC_BLK), 0)
    oh = jnp.where(ids == motif_ref[...][None, :],
                   smask_ref[...][None, :], 0.0)
    part = jnp.dot(oh, feats_ref[...], preferred_element_type=jnp.float32)

    @pl.when(i == 0)
    def _():
        out_ref[...] = part

    @pl.when(i > 0)
    def _():
        out_ref[...] += part


def _tc_readout(partials_ref, tc_part_ref, wf_ref, bf_ref, wo_ref, bo_ref,
                out_ref):
    seg = (jnp.sum(partials_ref[...], axis=0) + tc_part_ref[...])[1:]
    h = jnp.dot(seg, wf_ref[...], preferred_element_type=jnp.float32)
    h = h + bf_ref[...]
    o = jnp.dot(h, wo_ref[...], preferred_element_type=jnp.float32)
    out_ref[...] = o + bo_ref[...]


def kernel(node_feats, smask_full, motif_batch, W_feat, b_feat, W_out, b_out):
    partials = _sc_call(node_feats, smask_full, motif_batch)
    tc_part = pl.pallas_call(
        _tc_onehot_partial,
        grid=(N_TC // TC_BLK,),
        in_specs=[
            pl.BlockSpec((TC_BLK, D), lambda i: (i, 0)),
            pl.BlockSpec((TC_BLK,), lambda i: (i,)),
            pl.BlockSpec((TC_BLK,), lambda i: (i,)),
        ],
        out_specs=pl.BlockSpec((NUM_MOTIFS, D), lambda i: (0, 0)),
        out_shape=jax.ShapeDtypeStruct((NUM_MOTIFS, D), jnp.float32),
    )(node_feats, smask_full, motif_batch)
    return pl.pallas_call(
        _tc_readout,
        out_shape=jax.ShapeDtypeStruct((NUM_MOTIFS - 1, D_OUT), jnp.float32),
    )(partials, tc_part, W_feat, b_feat.reshape(1, D_HID), W_out,
      b_out.reshape(1, D_OUT))


# speedup vs baseline: 1.0707x; 1.0096x over previous
import functools

import jax
import jax.numpy as jnp
from jax import lax
from jax.experimental import pallas as pl
from jax.experimental.pallas import tpu as pltpu
from jax.experimental.pallas import tpu_sc as plsc

N = 32768
D = 256
D_HID = 512
D_OUT = 128
NUM_MOTIFS = 64

NC = 2
NS = 16
NW = NC * NS
L = 16
NJ = D // L

N_TC = 28672
N_SC = N - N_TC
ROWS_PER_W = N_SC // NW
CHUNK = 128
NCHUNK = ROWS_PER_W // CHUNK
GPC = CHUNK // L


def _sc_segment_sum(feats_hbm, smask_hbm, motif_hbm, out_hbm,
                    buf, smask_v, motif_v, acc, run, cur_m_ref, sem0):
    cid = lax.axis_index("c")
    sid = lax.axis_index("s")
    wid = sid * NC + cid
    base = N_TC + wid * ROWS_PER_W

    pltpu.async_copy(feats_hbm.at[pl.ds(base, CHUNK), :],
                     buf.at[pl.ds(0, CHUNK), :], sem0)
    pltpu.sync_copy(smask_hbm.at[pl.ds(base, ROWS_PER_W)],
                    smask_v.at[pl.ds(0, ROWS_PER_W)])
    pltpu.sync_copy(motif_hbm.at[pl.ds(base, ROWS_PER_W)],
                    motif_v.at[pl.ds(0, ROWS_PER_W)])

    zeros = jnp.zeros((L,), jnp.float32)
    for j in range(NJ):
        run[pl.ds(j * L, L)] = zeros

    def zero_row(i):
        for j in range(NJ):
            acc[i, pl.ds(j * L, L)] = zeros
    plsc.parallel_loop(0, NUM_MOTIFS)(zero_row)

    cur_m_ref[0] = motif_v[pl.ds(0, L)][0]

    def accumulate_rows(roff, goff, nrows):
        init = tuple(zeros for _ in range(NJ))

        def row_body(r, regs):
            sval = smask_v[pl.ds(goff + r, L)]
            s = sval[0]
            return tuple(
                regs[j] + buf[roff + r, pl.ds(j * L, L)] * s
                for j in range(NJ))

        regs = plsc.parallel_loop(0, nrows, carry=init)(row_body)
        for j in range(NJ):
            plsc.addupdate(run.at[pl.ds(j * L, L)], regs[j])

    def process_chunk(roff, coff):
        chunk_last = motif_v[pl.ds(coff + CHUNK - L, L)][L - 1]
        chunk_fast = chunk_last == cur_m_ref[0]

        @pl.when(chunk_fast)
        def _chunk_fast():
            accumulate_rows(roff, coff, CHUNK)

        @pl.when(jnp.logical_not(chunk_fast))
        def _chunk_slow():
            def group_body(g, gc):
                goff = coff + g * L
                mv = motif_v[pl.ds(goff, L)]
                group_fast = mv[L - 1] == cur_m_ref[0]

                @pl.when(group_fast)
                def _group_fast():
                    accumulate_rows(roff + g * L, goff, L)

                @pl.when(jnp.logical_not(group_fast))
                def _group_slow():
                    def row_slow(r, rc):
                        m = motif_v[pl.ds(goff + r, L)][0]
                        s = smask_v[pl.ds(goff + r, L)][0]
                        row = roff + g * L + r

                        @pl.when(m != cur_m_ref[0])
                        def _flush():
                            cm = cur_m_ref[0]
                            for j in range(NJ):
                                acc[cm, pl.ds(j * L, L)] = (
                                    run[pl.ds(j * L, L)])
                                run[pl.ds(j * L, L)] = zeros
                            cur_m_ref[0] = m

                        for j in range(NJ):
                            plsc.addupdate(
                                run.at[pl.ds(j * L, L)],
                                buf[row, pl.ds(j * L, L)] * s)
                        return rc
                    lax.fori_loop(0, L, row_slow, 0)
                return gc
            lax.fori_loop(0, GPC, group_body, 0)

    pltpu.make_async_copy(feats_hbm.at[pl.ds(base, CHUNK), :],
                          buf.at[pl.ds(0, CHUNK), :], sem0).wait()
    process_chunk(0, 0)

    cm = cur_m_ref[0]
    for j in range(NJ):
        acc[cm, pl.ds(j * L, L)] = run[pl.ds(j * L, L)]

    pltpu.sync_copy(acc, out_hbm.at[wid])


_sc_call = functools.partial(
    pl.kernel,
    mesh=plsc.VectorSubcoreMesh(core_axis_name="c", subcore_axis_name="s"),
    out_type=jax.ShapeDtypeStruct((NW, NUM_MOTIFS, D), jnp.float32),
    scratch_types=[
        pltpu.VMEM((CHUNK, D), jnp.float32),
        pltpu.VMEM((ROWS_PER_W + L,), jnp.float32),
        pltpu.VMEM((ROWS_PER_W + L,), jnp.int32),
        pltpu.VMEM((NUM_MOTIFS, D), jnp.float32),
        pltpu.VMEM((D,), jnp.float32),
        pltpu.SMEM((1,), jnp.int32),
        pltpu.SemaphoreType.DMA,
    ],
)(_sc_segment_sum)


TC_BLK = 7168


def _tc_onehot_partial(feats_ref, smask_ref, motif_ref, out_ref):
    i = pl.program_id(0)
    ids = lax.broadcasted_iota(jnp.int32, (NUM_MOTIFS, TC_BLK), 0)
    oh = jnp.where(ids == motif_ref[...][None, :],
                   smask_ref[...][None, :], 0.0)
    part = jnp.dot(oh, feats_ref[...], preferred_element_type=jnp.float32)

    @pl.when(i == 0)
    def _():
        out_ref[...] = part

    @pl.when(i > 0)
    def _():
        out_ref[...] += part


def _tc_readout(partials_ref, tc_part_ref, wf_ref, bf_ref, wo_ref, bo_ref,
                out_ref):
    seg = (jnp.sum(partials_ref[...], axis=0) + tc_part_ref[...])[1:]
    h = jnp.dot(seg, wf_ref[...], preferred_element_type=jnp.float32)
    h = h + bf_ref[...]
    o = jnp.dot(h, wo_ref[...], preferred_element_type=jnp.float32)
    out_ref[...] = o + bo_ref[...]


def kernel(node_feats, smask_full, motif_batch, W_feat, b_feat, W_out, b_out):
    partials = _sc_call(node_feats, smask_full, motif_batch)
    tc_part = pl.pallas_call(
        _tc_onehot_partial,
        grid=(N_TC // TC_BLK,),
        in_specs=[
            pl.BlockSpec((TC_BLK, D), lambda i: (i, 0)),
            pl.BlockSpec((TC_BLK,), lambda i: (i,)),
            pl.BlockSpec((TC_BLK,), lambda i: (i,)),
        ],
        out_specs=pl.BlockSpec((NUM_MOTIFS, D), lambda i: (0, 0)),
        out_shape=jax.ShapeDtypeStruct((NUM_MOTIFS, D), jnp.float32),
    )(node_feats, smask_full, motif_batch)
    return pl.pallas_call(
        _tc_readout,
        out_shape=jax.ShapeDtypeStruct((NUM_MOTIFS - 1, D_OUT), jnp.float32),
    )(partials, tc_part, W_feat, b_feat.reshape(1, D_HID), W_out,
      b_out.reshape(1, D_OUT))
